# Initial kernel scaffold; baseline (speedup 1.0000x reference)
import jax
import jax.numpy as jnp
from jax.experimental import pallas as pl


def kernel(x, gmf_t0, gmf_t1, mlp_t0, mlp_t1, w1, b1, w2, b2, wfc, bfc):
    raise NotImplementedError("write your pallas kernel here")


# speedup vs baseline: 4.6399x; 4.6399x over previous
import functools

import jax
import jax.numpy as jnp
from jax import lax
from jax.experimental import pallas as pl
from jax.experimental.pallas import tpu as pltpu

_TILE = 512
_UNROLL = 8


def _round_up(n, m):
    return ((n + m - 1) // m) * m


def _ncf_body(idx_ref,
              t0_ref, t1_ref,
              w1a_ref, w1b_ref,
              b1_ref, w2_ref, b2_ref,
              wg_ref, wm_ref,
              bfc_ref,
              out_ref,
              a0, a1):
    base = pl.program_id(0) * _TILE

    def gather_chunk(c, carry):
        row = c * _UNROLL
        for j in range(_UNROLL):
            m = row + j
            a0[m] = t0_ref[idx_ref[0, base + m], 0]
            a1[m] = t1_ref[idx_ref[1, base + m], 0]
        return carry

    lax.fori_loop(0, _TILE // _UNROLL, gather_chunk, 0)

    A0 = a0[...]
    A1 = a1[...]
    prod = A0 * A1

    h = (jnp.dot(A0, w1a_ref[...], preferred_element_type=jnp.float32)
         + jnp.dot(A1, w1b_ref[...], preferred_element_type=jnp.float32)
         + b1_ref[...])
    h = jnp.maximum(h, 0.0)
    h = jnp.dot(h, w2_ref[...], preferred_element_type=jnp.float32) + b2_ref[...]
    h = jnp.maximum(h, 0.0)

    logit = (jnp.sum(prod * wg_ref[...], axis=-1, keepdims=True)
             + jnp.sum(h * wm_ref[...], axis=-1, keepdims=True)
             + bfc_ref[0, 0])
    out_ref[...] = jax.nn.sigmoid(logit)


@functools.partial(jax.jit, static_argnames=())
def kernel(x, gmf_t0, gmf_t1, mlp_t0, mlp_t1, w1, b1, w2, b2, wfc, bfc):
    B = x.shape[0]
    E = gmf_t0.shape[1]
    D = 2 * E

    b_pad = _round_up(max(B, 1), _TILE)
    num_tiles = b_pad // _TILE

    idx = x.astype(jnp.int32).T
    if b_pad != B:
        idx = jnp.pad(idx, ((0, 0), (0, b_pad - B)))

    t0 = jnp.concatenate([gmf_t0, mlp_t0], axis=1).reshape(-1, 1, D)
    t1 = jnp.concatenate([gmf_t1, mlp_t1], axis=1).reshape(-1, 1, D)

    zeros_top = jnp.zeros((E, 128), jnp.float32)
    w1a = jnp.concatenate([zeros_top, w1[:E, :]], axis=0)
    w1b = jnp.concatenate([zeros_top, w1[E:, :]], axis=0)
    wg = jnp.pad(wfc[:E, :].T, ((0, 0), (0, D - E)))
    wm = wfc[E:, :].T

    def resident(a):
        return pl.BlockSpec(a.shape, lambda g, s: (0,) * a.ndim)

    grid_spec = pltpu.PrefetchScalarGridSpec(
        num_scalar_prefetch=1,
        grid=(num_tiles,),
        in_specs=[
            resident(t0), resident(t1),
            resident(w1a), resident(w1b), resident(b1),
            resident(w2), resident(b2),
            resident(wg), resident(wm),
            pl.BlockSpec(memory_space=pltpu.MemorySpace.SMEM),
        ],
        out_specs=pl.BlockSpec((_TILE, 1), lambda g, s: (g, 0)),
        scratch_shapes=[
            pltpu.VMEM((_TILE, D), jnp.float32),
            pltpu.VMEM((_TILE, D), jnp.float32),
        ],
    )

    flops = 2 * b_pad * (D * 128 * 2 + 128 * 64) + b_pad * (4 * D + 4 * 64)
    bytes_accessed = (t0.size + t1.size) * 4 + b_pad * (2 * 4 + D * 8 + 4)
    out = pl.pallas_call(
        _ncf_body,
        out_shape=jax.ShapeDtypeStruct((b_pad, 1), jnp.float32),
        grid_spec=grid_spec,
        compiler_params=pltpu.CompilerParams(
            dimension_semantics=("parallel",)),
        cost_estimate=pl.CostEstimate(flops=flops, transcendentals=b_pad,
                                      bytes_accessed=bytes_accessed),
    )(idx, t0, t1, w1a, w1b, b1, w2, b2, wg, wm, bfc)
    return out[:B]


# speedup vs baseline: 5.6441x; 1.2164x over previous
import functools

import jax
import jax.numpy as jnp
from jax import lax
from jax.experimental import pallas as pl
from jax.experimental.pallas import tpu as pltpu

_TILE = 256


def _round_up(n, m):
    return ((n + m - 1) // m) * m


def _ncf_body(idx_ref,
              t0_ref, t1_ref,
              w1a_ref, w1b_ref,
              b1_ref, w2_ref, b2_ref,
              wg_ref, wm_ref,
              bfc_ref,
              out_ref,
              a0, a1):
    for m in range(_TILE):
        a0[m] = t0_ref[idx_ref[0, m], 0]
        a1[m] = t1_ref[idx_ref[1, m], 0]

    A0 = a0[...]
    A1 = a1[...]
    prod = A0 * A1

    h = (jnp.dot(A0, w1a_ref[...], preferred_element_type=jnp.float32)
         + jnp.dot(A1, w1b_ref[...], preferred_element_type=jnp.float32)
         + b1_ref[...])
    h = jnp.maximum(h, 0.0)
    h = jnp.dot(h, w2_ref[...], preferred_element_type=jnp.float32) + b2_ref[...]
    h = jnp.maximum(h, 0.0)

    logit = (jnp.sum(prod * wg_ref[...], axis=-1, keepdims=True)
             + jnp.sum(h * wm_ref[...], axis=-1, keepdims=True)
             + bfc_ref[0, 0])
    out_ref[...] = jax.nn.sigmoid(logit)


@functools.partial(jax.jit, static_argnames=())
def kernel(x, gmf_t0, gmf_t1, mlp_t0, mlp_t1, w1, b1, w2, b2, wfc, bfc):
    B = x.shape[0]
    E = gmf_t0.shape[1]
    D = 2 * E

    b_pad = _round_up(max(B, 1), _TILE)
    num_tiles = b_pad // _TILE

    idx = x.astype(jnp.int32).T
    if b_pad != B:
        idx = jnp.pad(idx, ((0, 0), (0, b_pad - B)))

    t0 = jnp.concatenate([gmf_t0, mlp_t0], axis=1).reshape(-1, 1, D)
    t1 = jnp.concatenate([gmf_t1, mlp_t1], axis=1).reshape(-1, 1, D)

    zeros_top = jnp.zeros((E, 128), jnp.float32)
    w1a = jnp.concatenate([zeros_top, w1[:E, :]], axis=0)
    w1b = jnp.concatenate([zeros_top, w1[E:, :]], axis=0)
    wg = jnp.pad(wfc[:E, :].T, ((0, 0), (0, D - E)))
    wm = wfc[E:, :].T

    def resident(a):
        return pl.BlockSpec(a.shape, lambda g: (0,) * a.ndim)

    flops = 2 * b_pad * (D * 128 * 2 + 128 * 64) + b_pad * (4 * D + 4 * 64)
    bytes_accessed = (t0.size + t1.size) * 4 + b_pad * (2 * 4 + D * 8 + 4)
    out = pl.pallas_call(
        _ncf_body,
        out_shape=jax.ShapeDtypeStruct((b_pad, 1), jnp.float32),
        grid=(num_tiles,),
        in_specs=[
            pl.BlockSpec((2, _TILE), lambda g: (0, g),
                         memory_space=pltpu.MemorySpace.SMEM),
            resident(t0), resident(t1),
            resident(w1a), resident(w1b), resident(b1),
            resident(w2), resident(b2),
            resident(wg), resident(wm),
            pl.BlockSpec(memory_space=pltpu.MemorySpace.SMEM),
        ],
        out_specs=pl.BlockSpec((_TILE, 1), lambda g: (g, 0)),
        scratch_shapes=[
            pltpu.VMEM((_TILE, D), jnp.float32),
            pltpu.VMEM((_TILE, D), jnp.float32),
        ],
        compiler_params=pltpu.CompilerParams(
            dimension_semantics=("parallel",)),
        cost_estimate=pl.CostEstimate(flops=flops, transcendentals=b_pad,
                                      bytes_accessed=bytes_accessed),
    )(idx, t0, t1, w1a, w1b, b1, w2, b2, wg, wm, bfc)
    return out[:B]
